# Initial kernel scaffold; baseline (speedup 1.0000x reference)
import jax
import jax.numpy as jnp
from jax.experimental import pallas as pl


def kernel(edge_index, user_indices, job_indices, user_emb, job_emb, W1, b1, W2, b2, Wp, bp):
    raise NotImplementedError("write your pallas kernel here")


# speedup vs baseline: 11.0183x; 11.0183x over previous
import functools

import jax
import jax.numpy as jnp
from jax import lax
from jax.experimental import pallas as pl
from jax.experimental.pallas import tpu as pltpu
from jax.experimental.pallas import tpu_sc as plsc

NUM_USERS = 5000
N = 10000
D = 128
E = 320000
B = 16384

NC = 2
NS = 16
NW = NC * NS
EPW = E // NW
CH = 80
NCH = EPW // CH
NPAD = 10240
RPT = NPAD // NS
BPW = B // NW
PCH = 128
NPC = BPW // PCH


def _mesh():
    return plsc.VectorSubcoreMesh(
        core_axis_name="c", subcore_axis_name="s", num_cores=NC, num_subcores=NS
    )


def _sc_degree(dst_flat, ones_row, zrow):
    @functools.partial(
        pl.kernel,
        mesh=_mesh(),
        out_type=jax.ShapeDtypeStruct((NC, NPAD, D), jnp.float32),
        scratch_types=[
            pltpu.VMEM((CH,), jnp.int32),
            pltpu.VMEM((CH, D), jnp.float32),
            pltpu.VMEM_SHARED((NPAD, D), jnp.float32),
        ],
    )
    def k(dst_hbm, ones_hbm, z_hbm, out_hbm, idx_v, buf, acc_sh):
        cid = lax.axis_index("c")
        sid = lax.axis_index("s")
        wid = cid * NS + sid
        pltpu.sync_copy(z_hbm, buf)
        for t in range(RPT // CH):
            pltpu.sync_copy(buf, acc_sh.at[pl.ds(sid * RPT + t * CH, CH)])
        pltpu.sync_copy(ones_hbm, buf)
        plsc.subcore_barrier()

        def body(j, carry):
            pltpu.sync_copy(dst_hbm.at[pl.ds(wid * EPW + j * CH, CH)], idx_v)
            pltpu.sync_copy(buf, acc_sh.at[idx_v], add=True)
            return carry

        lax.fori_loop(0, NCH, body, 0)
        plsc.subcore_barrier()
        for t in range(RPT // CH):
            pltpu.sync_copy(acc_sh.at[pl.ds(sid * RPT + t * CH, CH)], buf)
            pltpu.sync_copy(buf, out_hbm.at[cid, pl.ds(sid * RPT + t * CH, CH)])

    return k(dst_flat, ones_row, zrow)


def _sc_edge_agg(hs, src_flat, dst_flat, zrow):
    @functools.partial(
        pl.kernel,
        mesh=_mesh(),
        out_type=jax.ShapeDtypeStruct((NC, NPAD, D), jnp.float32),
        scratch_types=[
            pltpu.VMEM((CH,), jnp.int32),
            pltpu.VMEM((CH,), jnp.int32),
            pltpu.VMEM((CH, D), jnp.float32),
            pltpu.VMEM_SHARED((NPAD, D), jnp.float32),
            pltpu.SemaphoreType.DMA,
        ],
    )
    def k(hs_hbm, src_hbm, dst_hbm, z_hbm, out_hbm, sidx, didx, rows, acc_sh, sem):
        cid = lax.axis_index("c")
        sid = lax.axis_index("s")
        wid = cid * NS + sid
        pltpu.sync_copy(z_hbm, rows)
        for t in range(RPT // CH):
            pltpu.sync_copy(rows, acc_sh.at[pl.ds(sid * RPT + t * CH, CH)])
        plsc.subcore_barrier()

        def body(j, carry):
            pltpu.sync_copy(src_hbm.at[pl.ds(wid * EPW + j * CH, CH)], sidx)
            pltpu.sync_copy(dst_hbm.at[pl.ds(wid * EPW + j * CH, CH)], didx)
            pltpu.async_copy(hs_hbm.at[sidx], rows, sem).wait()
            pltpu.sync_copy(rows, acc_sh.at[didx], add=True)
            return carry

        lax.fori_loop(0, NCH, body, 0)
        plsc.subcore_barrier()
        for t in range(RPT // CH):
            pltpu.sync_copy(acc_sh.at[pl.ds(sid * RPT + t * CH, CH)], rows)
            pltpu.sync_copy(rows, out_hbm.at[cid, pl.ds(sid * RPT + t * CH, CH)])

    return k(hs, src_flat, dst_flat, zrow)


def _sc_predict(x2, ui_flat, ji_flat):
    @functools.partial(
        pl.kernel,
        mesh=_mesh(),
        out_type=(
            jax.ShapeDtypeStruct((B, D), jnp.float32),
            jax.ShapeDtypeStruct((B, D), jnp.float32),
        ),
        scratch_types=[
            pltpu.VMEM((PCH,), jnp.int32),
            pltpu.VMEM((PCH,), jnp.int32),
            pltpu.VMEM((PCH, D), jnp.float32),
            pltpu.VMEM((PCH, D), jnp.float32),
            pltpu.SemaphoreType.DMA,
            pltpu.SemaphoreType.DMA,
        ],
    )
    def k(x2_hbm, ui_hbm, ji_hbm, gu_hbm, gj_hbm, uix, jix, ub, jb, s0, s1):
        cid = lax.axis_index("c")
        sid = lax.axis_index("s")
        wid = cid * NS + sid

        def body(t, carry):
            pltpu.sync_copy(ui_hbm.at[pl.ds(wid * BPW + t * PCH, PCH)], uix)
            pltpu.sync_copy(ji_hbm.at[pl.ds(wid * BPW + t * PCH, PCH)], jix)
            cu = pltpu.async_copy(x2_hbm.at[uix], ub, s0)
            cj = pltpu.async_copy(x2_hbm.at[jix], jb, s1)
            cu.wait()
            cj.wait()
            pltpu.sync_copy(ub, gu_hbm.at[pl.ds(wid * BPW + t * PCH, PCH)])
            pltpu.sync_copy(jb, gj_hbm.at[pl.ds(wid * BPW + t * PCH, PCH)])
            return carry

        lax.fori_loop(0, NPC, body, 0)

    return k(x2, ui_flat, ji_flat)


def _tc_first(x, W1, degp):
    def body(x_ref, w_ref, degp_ref, hs_ref, dis_ref):
        deg = degp_ref[0, :N, :1] + degp_ref[1, :N, :1] + 1.0
        dis = lax.rsqrt(deg)
        h = jnp.dot(x_ref[...], w_ref[...], preferred_element_type=jnp.float32)
        hs_ref[...] = h * dis
        dis_ref[...] = dis

    return pl.pallas_call(
        body,
        out_shape=(
            jax.ShapeDtypeStruct((N, D), jnp.float32),
            jax.ShapeDtypeStruct((N, 1), jnp.float32),
        ),
    )(x, W1, degp)


def _tc_mid(aggp, hs1, dis, b1, W2):
    def body(aggp_ref, hs_ref, dis_ref, b_ref, w_ref, out_ref):
        agg = aggp_ref[0, :N] + aggp_ref[1, :N] + hs_ref[...]
        x1 = jnp.maximum(agg * dis_ref[...] + b_ref[...], 0.0)
        out_ref[...] = (
            jnp.dot(x1, w_ref[...], preferred_element_type=jnp.float32) * dis_ref[...]
        )

    return pl.pallas_call(
        body, out_shape=jax.ShapeDtypeStruct((N, D), jnp.float32)
    )(aggp, hs1, dis, b1, W2)


def _tc_final(aggp, hs2, dis, b2):
    def body(aggp_ref, hs_ref, dis_ref, b_ref, out_ref):
        out_ref[...] = (
            aggp_ref[0, :N] + aggp_ref[1, :N] + hs_ref[...]
        ) * dis_ref[...] + b_ref[...]

    return pl.pallas_call(
        body, out_shape=jax.ShapeDtypeStruct((N, D), jnp.float32)
    )(aggp, hs2, dis, b2)


def _tc_addpred(gu, gj, wpu, wpj):
    def body(gu_ref, gj_ref, wu_ref, wj_ref, out_ref):
        t = gu_ref[...] * wu_ref[...] + gj_ref[...] * wj_ref[...]
        out_ref[...] = jnp.sum(t, axis=1, keepdims=True)

    return pl.pallas_call(
        body, out_shape=jax.ShapeDtypeStruct((B, 1), jnp.float32)
    )(gu, gj, wpu, wpj)


def kernel(edge_index, user_indices, job_indices, user_emb, job_emb, W1, b1, W2, b2, Wp, bp):
    f32 = jnp.float32
    x = jnp.concatenate([user_emb, job_emb], axis=0)
    src_flat = edge_index[0]
    dst_flat = edge_index[1]
    ones_row = jnp.ones((CH, D), f32)
    zrow = jnp.zeros((CH, D), f32)

    degp = _sc_degree(dst_flat, ones_row, zrow)
    hs1, dis = _tc_first(x, W1, degp)
    aggp1 = _sc_edge_agg(hs1, src_flat, dst_flat, zrow)
    hs2 = _tc_mid(aggp1, hs1, dis, b1.reshape(1, D), W2)
    aggp2 = _sc_edge_agg(hs2, src_flat, dst_flat, zrow)
    x2 = _tc_final(aggp2, hs2, dis, b2.reshape(1, D))
    gu, gj = _sc_predict(x2, user_indices, job_indices + NUM_USERS)
    wpu = Wp[:D, 0].reshape(1, D)
    wpj = Wp[D:, 0].reshape(1, D)
    pred = _tc_addpred(gu, gj, wpu, wpj)
    return pred.reshape(B) + bp[0]


# speedup vs baseline: 17.1817x; 1.5594x over previous
import functools

import jax
import jax.numpy as jnp
from jax import lax
from jax.experimental import pallas as pl
from jax.experimental.pallas import tpu as pltpu
from jax.experimental.pallas import tpu_sc as plsc

NUM_USERS = 5000
N = 10000
D = 128
E = 320000
B = 16384

NC = 2
NS = 16
NW = NC * NS
EPW = E // NW
CH = 80
NCH = EPW // CH
NPAD = 10240
RPT = NPAD // NS
BPW = B // NW
PCH = 128
NPC = BPW // PCH


def _mesh():
    return plsc.VectorSubcoreMesh(
        core_axis_name="c", subcore_axis_name="s", num_cores=NC, num_subcores=NS
    )


def _sc_degree(dst_flat, ones_row, zrow):
    @functools.partial(
        pl.kernel,
        mesh=_mesh(),
        out_type=jax.ShapeDtypeStruct((NC, NPAD, D), jnp.float32),
        scratch_types=[
            pltpu.VMEM((CH,), jnp.int32),
            pltpu.VMEM((CH,), jnp.int32),
            pltpu.VMEM((CH, D), jnp.float32),
            pltpu.VMEM_SHARED((NPAD, D), jnp.float32),
            pltpu.SemaphoreType.DMA,
        ],
    )
    def k(dst_hbm, ones_hbm, z_hbm, out_hbm, idx0, idx1, buf, acc_sh, sem):
        cid = lax.axis_index("c")
        sid = lax.axis_index("s")
        wid = cid * NS + sid
        base = wid * EPW
        pltpu.sync_copy(z_hbm, buf)
        for t in range(RPT // CH):
            pltpu.sync_copy(buf, acc_sh.at[pl.ds(sid * RPT + t * CH, CH)])
        pltpu.sync_copy(ones_hbm, buf)
        plsc.subcore_barrier()
        pltpu.sync_copy(dst_hbm.at[pl.ds(base, CH)], idx0)

        def body(i, carry):
            j1 = 2 * i + 1
            j2 = jnp.minimum(2 * i + 2, NCH - 1)
            pltpu.async_copy(dst_hbm.at[pl.ds(base + j1 * CH, CH)], idx1, sem)
            pltpu.sync_copy(buf, acc_sh.at[idx0], add=True)
            pltpu.make_async_copy(dst_hbm.at[pl.ds(base, CH)], idx1, sem).wait()
            pltpu.async_copy(dst_hbm.at[pl.ds(base + j2 * CH, CH)], idx0, sem)
            pltpu.sync_copy(buf, acc_sh.at[idx1], add=True)
            pltpu.make_async_copy(dst_hbm.at[pl.ds(base, CH)], idx0, sem).wait()
            return carry

        lax.fori_loop(0, (NCH - 1) // 2, body, 0)
        pltpu.sync_copy(buf, acc_sh.at[idx0], add=True)
        plsc.subcore_barrier()
        for t in range(RPT // CH):
            pltpu.sync_copy(acc_sh.at[pl.ds(sid * RPT + t * CH, CH)], buf)
            pltpu.sync_copy(buf, out_hbm.at[cid, pl.ds(sid * RPT + t * CH, CH)])

    return k(dst_flat, ones_row, zrow)


def _sc_edge_agg(hs, src_flat, dst_flat, zrow):
    @functools.partial(
        pl.kernel,
        mesh=_mesh(),
        out_type=jax.ShapeDtypeStruct((NC, NPAD, D), jnp.float32),
        scratch_types=[
            pltpu.VMEM((CH,), jnp.int32),
            pltpu.VMEM((CH,), jnp.int32),
            pltpu.VMEM((CH,), jnp.int32),
            pltpu.VMEM((CH,), jnp.int32),
            pltpu.VMEM((CH, D), jnp.float32),
            pltpu.VMEM((CH, D), jnp.float32),
            pltpu.VMEM_SHARED((NPAD, D), jnp.float32),
            pltpu.SemaphoreType.DMA,
            pltpu.SemaphoreType.DMA,
        ],
    )
    def k(hs_hbm, src_hbm, dst_hbm, z_hbm, out_hbm,
          sidx0, didx0, sidx1, didx1, rows0, rows1, acc_sh, sem0, sem1):
        cid = lax.axis_index("c")
        sid = lax.axis_index("s")
        wid = cid * NS + sid
        base = wid * EPW
        pltpu.sync_copy(z_hbm, rows0)
        for t in range(RPT // CH):
            pltpu.sync_copy(rows0, acc_sh.at[pl.ds(sid * RPT + t * CH, CH)])
        plsc.subcore_barrier()
        pltpu.sync_copy(src_hbm.at[pl.ds(base, CH)], sidx0)
        pltpu.sync_copy(dst_hbm.at[pl.ds(base, CH)], didx0)
        pltpu.async_copy(hs_hbm.at[sidx0], rows0, sem0)

        def body(i, carry):
            j1 = 2 * i + 1
            j2 = jnp.minimum(2 * i + 2, NCH - 1)
            pltpu.sync_copy(src_hbm.at[pl.ds(base + j1 * CH, CH)], sidx1)
            pltpu.sync_copy(dst_hbm.at[pl.ds(base + j1 * CH, CH)], didx1)
            pltpu.async_copy(hs_hbm.at[sidx1], rows1, sem1)
            pltpu.make_async_copy(hs_hbm.at[sidx0], rows0, sem0).wait()
            pltpu.sync_copy(rows0, acc_sh.at[didx0], add=True)
            pltpu.sync_copy(src_hbm.at[pl.ds(base + j2 * CH, CH)], sidx0)
            pltpu.sync_copy(dst_hbm.at[pl.ds(base + j2 * CH, CH)], didx0)
            pltpu.async_copy(hs_hbm.at[sidx0], rows0, sem0)
            pltpu.make_async_copy(hs_hbm.at[sidx1], rows1, sem1).wait()
            pltpu.sync_copy(rows1, acc_sh.at[didx1], add=True)
            return carry

        lax.fori_loop(0, (NCH - 1) // 2, body, 0)
        pltpu.make_async_copy(hs_hbm.at[sidx0], rows0, sem0).wait()
        pltpu.sync_copy(rows0, acc_sh.at[didx0], add=True)
        plsc.subcore_barrier()
        for t in range(RPT // CH):
            pltpu.sync_copy(acc_sh.at[pl.ds(sid * RPT + t * CH, CH)], rows0)
            pltpu.sync_copy(rows0, out_hbm.at[cid, pl.ds(sid * RPT + t * CH, CH)])

    return k(hs, src_flat, dst_flat, zrow)


def _sc_predict(x2, ui_flat, ji_flat):
    @functools.partial(
        pl.kernel,
        mesh=_mesh(),
        out_type=(
            jax.ShapeDtypeStruct((B, D), jnp.float32),
            jax.ShapeDtypeStruct((B, D), jnp.float32),
        ),
        scratch_types=[
            pltpu.VMEM((PCH,), jnp.int32),
            pltpu.VMEM((PCH,), jnp.int32),
            pltpu.VMEM((PCH, D), jnp.float32),
            pltpu.VMEM((PCH, D), jnp.float32),
            pltpu.SemaphoreType.DMA,
            pltpu.SemaphoreType.DMA,
        ],
    )
    def k(x2_hbm, ui_hbm, ji_hbm, gu_hbm, gj_hbm, uix, jix, ub, jb, s0, s1):
        cid = lax.axis_index("c")
        sid = lax.axis_index("s")
        wid = cid * NS + sid

        def body(t, carry):
            pltpu.sync_copy(ui_hbm.at[pl.ds(wid * BPW + t * PCH, PCH)], uix)
            pltpu.sync_copy(ji_hbm.at[pl.ds(wid * BPW + t * PCH, PCH)], jix)
            cu = pltpu.async_copy(x2_hbm.at[uix], ub, s0)
            cj = pltpu.async_copy(x2_hbm.at[jix], jb, s1)
            cu.wait()
            cj.wait()
            pltpu.sync_copy(ub, gu_hbm.at[pl.ds(wid * BPW + t * PCH, PCH)])
            pltpu.sync_copy(jb, gj_hbm.at[pl.ds(wid * BPW + t * PCH, PCH)])
            return carry

        lax.fori_loop(0, NPC, body, 0)

    return k(x2, ui_flat, ji_flat)


def _tc_first(x, W1, degp):
    def body(x_ref, w_ref, degp_ref, hs_ref, dis_ref):
        deg = degp_ref[0, :N, :1] + degp_ref[1, :N, :1] + 1.0
        dis = lax.rsqrt(deg)
        h = jnp.dot(x_ref[...], w_ref[...], preferred_element_type=jnp.float32)
        hs_ref[...] = h * dis
        dis_ref[...] = dis

    return pl.pallas_call(
        body,
        out_shape=(
            jax.ShapeDtypeStruct((N, D), jnp.float32),
            jax.ShapeDtypeStruct((N, 1), jnp.float32),
        ),
    )(x, W1, degp)


def _tc_mid(aggp, hs1, dis, b1, W2):
    def body(aggp_ref, hs_ref, dis_ref, b_ref, w_ref, out_ref):
        agg = aggp_ref[0, :N] + aggp_ref[1, :N] + hs_ref[...]
        x1 = jnp.maximum(agg * dis_ref[...] + b_ref[...], 0.0)
        out_ref[...] = (
            jnp.dot(x1, w_ref[...], preferred_element_type=jnp.float32) * dis_ref[...]
        )

    return pl.pallas_call(
        body, out_shape=jax.ShapeDtypeStruct((N, D), jnp.float32)
    )(aggp, hs1, dis, b1, W2)


def _tc_final(aggp, hs2, dis, b2):
    def body(aggp_ref, hs_ref, dis_ref, b_ref, out_ref):
        out_ref[...] = (
            aggp_ref[0, :N] + aggp_ref[1, :N] + hs_ref[...]
        ) * dis_ref[...] + b_ref[...]

    return pl.pallas_call(
        body, out_shape=jax.ShapeDtypeStruct((N, D), jnp.float32)
    )(aggp, hs2, dis, b2)


def _tc_addpred(gu, gj, wpu, wpj):
    def body(gu_ref, gj_ref, wu_ref, wj_ref, out_ref):
        t = gu_ref[...] * wu_ref[...] + gj_ref[...] * wj_ref[...]
        out_ref[...] = jnp.sum(t, axis=1, keepdims=True)

    return pl.pallas_call(
        body, out_shape=jax.ShapeDtypeStruct((B, 1), jnp.float32)
    )(gu, gj, wpu, wpj)


def kernel(edge_index, user_indices, job_indices, user_emb, job_emb, W1, b1, W2, b2, Wp, bp):
    f32 = jnp.float32
    x = jnp.concatenate([user_emb, job_emb], axis=0)
    src_flat = edge_index[0]
    dst_flat = edge_index[1]
    ones_row = jnp.ones((CH, D), f32)
    zrow = jnp.zeros((CH, D), f32)

    degp = _sc_degree(dst_flat, ones_row, zrow)
    hs1, dis = _tc_first(x, W1, degp)
    aggp1 = _sc_edge_agg(hs1, src_flat, dst_flat, zrow)
    hs2 = _tc_mid(aggp1, hs1, dis, b1.reshape(1, D), W2)
    aggp2 = _sc_edge_agg(hs2, src_flat, dst_flat, zrow)
    x2 = _tc_final(aggp2, hs2, dis, b2.reshape(1, D))
    gu, gj = _sc_predict(x2, user_indices, job_indices + NUM_USERS)
    wpu = Wp[:D, 0].reshape(1, D)
    wpj = Wp[D:, 0].reshape(1, D)
    pred = _tc_addpred(gu, gj, wpu, wpj)
    return pred.reshape(B) + bp[0]


# speedup vs baseline: 22.0522x; 1.2835x over previous
import functools

import jax
import jax.numpy as jnp
from jax import lax
from jax.experimental import pallas as pl
from jax.experimental.pallas import tpu as pltpu
from jax.experimental.pallas import tpu_sc as plsc

NUM_USERS = 5000
N = 10000
D = 128
E = 320000
B = 16384

NC = 2
NS = 16
NW = NC * NS
EPW = E // NW
CH = 80
NCH = EPW // CH
NPAD = 10240
RPT = NPAD // NS
BPW = B // NW
PCH = 128
NPC = BPW // PCH


def _mesh():
    return plsc.VectorSubcoreMesh(
        core_axis_name="c", subcore_axis_name="s", num_cores=NC, num_subcores=NS
    )


def _sc_degree(dst_flat, ones_row, zrow):
    @functools.partial(
        pl.kernel,
        mesh=_mesh(),
        out_type=jax.ShapeDtypeStruct((NC, NPAD, D), jnp.float32),
        scratch_types=[
            pltpu.VMEM((CH,), jnp.int32),
            pltpu.VMEM((CH,), jnp.int32),
            pltpu.VMEM((CH, D), jnp.float32),
            pltpu.VMEM_SHARED((NPAD, D), jnp.float32),
            pltpu.SemaphoreType.DMA,
        ],
    )
    def k(dst_hbm, ones_hbm, z_hbm, out_hbm, idx0, idx1, buf, acc_sh, sem):
        cid = lax.axis_index("c")
        sid = lax.axis_index("s")
        wid = cid * NS + sid
        base = wid * EPW
        pltpu.sync_copy(z_hbm, buf)
        for t in range(RPT // CH):
            pltpu.sync_copy(buf, acc_sh.at[pl.ds(sid * RPT + t * CH, CH)])
        pltpu.sync_copy(ones_hbm, buf)
        plsc.subcore_barrier()
        pltpu.sync_copy(dst_hbm.at[pl.ds(base, CH)], idx0)

        def body(i, carry):
            j1 = 2 * i + 1
            j2 = jnp.minimum(2 * i + 2, NCH - 1)
            pltpu.async_copy(dst_hbm.at[pl.ds(base + j1 * CH, CH)], idx1, sem)
            pltpu.sync_copy(buf, acc_sh.at[idx0], add=True)
            pltpu.make_async_copy(dst_hbm.at[pl.ds(base, CH)], idx1, sem).wait()
            pltpu.async_copy(dst_hbm.at[pl.ds(base + j2 * CH, CH)], idx0, sem)
            pltpu.sync_copy(buf, acc_sh.at[idx1], add=True)
            pltpu.make_async_copy(dst_hbm.at[pl.ds(base, CH)], idx0, sem).wait()
            return carry

        lax.fori_loop(0, (NCH - 1) // 2, body, 0)
        pltpu.sync_copy(buf, acc_sh.at[idx0], add=True)
        plsc.subcore_barrier()
        for t in range(RPT // CH):
            pltpu.sync_copy(acc_sh.at[pl.ds(sid * RPT + t * CH, CH)], buf)
            pltpu.sync_copy(buf, out_hbm.at[cid, pl.ds(sid * RPT + t * CH, CH)])

    return k(dst_flat, ones_row, zrow)


def _sc_edge_agg(hs, src_flat, dst_flat, zrow):
    @functools.partial(
        pl.kernel,
        mesh=_mesh(),
        out_type=jax.ShapeDtypeStruct((NC, NPAD, D), jnp.float32),
        scratch_types=[
            [pltpu.VMEM((CH,), jnp.int32)] * 4,
            [pltpu.VMEM((CH,), jnp.int32)] * 4,
            pltpu.VMEM((CH, D), jnp.float32),
            pltpu.VMEM((CH, D), jnp.float32),
            pltpu.VMEM_SHARED((NPAD, D), jnp.float32),
            pltpu.SemaphoreType.DMA,
            pltpu.SemaphoreType.DMA,
            [pltpu.SemaphoreType.DMA] * 4,
        ],
    )
    def k(hs_hbm, src_hbm, dst_hbm, z_hbm, out_hbm,
          sidx, didx, rows0, rows1, acc_sh, sem0, sem1, semi):
        cid = lax.axis_index("c")
        sid = lax.axis_index("s")
        wid = cid * NS + sid
        base = wid * EPW

        def load_idx(j, k, sem=None):
            if sem is None:
                pltpu.sync_copy(src_hbm.at[pl.ds(base + j * CH, CH)], sidx[k])
                pltpu.sync_copy(dst_hbm.at[pl.ds(base + j * CH, CH)], didx[k])
            else:
                pltpu.async_copy(src_hbm.at[pl.ds(base + j * CH, CH)], sidx[k], sem)
                pltpu.async_copy(dst_hbm.at[pl.ds(base + j * CH, CH)], didx[k], sem)

        def wait_idx(k):
            pltpu.make_async_copy(src_hbm.at[pl.ds(base, CH)], sidx[k], semi[k]).wait()
            pltpu.make_async_copy(dst_hbm.at[pl.ds(base, CH)], didx[k], semi[k]).wait()

        def wait_rows(buf, sem):
            pltpu.make_async_copy(hs_hbm.at[sidx[0]], buf, sem).wait()

        pltpu.sync_copy(z_hbm, rows0)
        for t in range(RPT // CH):
            pltpu.sync_copy(rows0, acc_sh.at[pl.ds(sid * RPT + t * CH, CH)])
        plsc.subcore_barrier()
        load_idx(0, 0)
        load_idx(1, 1)
        load_idx(2, 2, semi[2])
        load_idx(3, 3, semi[3])
        pltpu.async_copy(hs_hbm.at[sidx[0]], rows0, sem0)
        pltpu.async_copy(hs_hbm.at[sidx[1]], rows1, sem1)

        def body(i, carry):
            j4 = jnp.minimum(4 * i + 4, NCH - 1)
            j5 = jnp.minimum(4 * i + 5, NCH - 1)
            j6 = jnp.minimum(4 * i + 6, NCH - 1)
            j7 = jnp.minimum(4 * i + 7, NCH - 1)
            wait_rows(rows0, sem0)
            pltpu.sync_copy(rows0, acc_sh.at[didx[0]], add=True)
            load_idx(j4, 0, semi[0])
            wait_idx(2)
            pltpu.async_copy(hs_hbm.at[sidx[2]], rows0, sem0)
            wait_rows(rows1, sem1)
            pltpu.sync_copy(rows1, acc_sh.at[didx[1]], add=True)
            load_idx(j5, 1, semi[1])
            wait_idx(3)
            pltpu.async_copy(hs_hbm.at[sidx[3]], rows1, sem1)
            wait_rows(rows0, sem0)
            pltpu.sync_copy(rows0, acc_sh.at[didx[2]], add=True)
            load_idx(j6, 2, semi[2])
            wait_idx(0)
            pltpu.async_copy(hs_hbm.at[sidx[0]], rows0, sem0)
            wait_rows(rows1, sem1)
            pltpu.sync_copy(rows1, acc_sh.at[didx[3]], add=True)
            load_idx(j7, 3, semi[3])
            wait_idx(1)
            pltpu.async_copy(hs_hbm.at[sidx[1]], rows1, sem1)
            return carry

        lax.fori_loop(0, (NCH - 1) // 4, body, 0)
        wait_rows(rows0, sem0)
        pltpu.sync_copy(rows0, acc_sh.at[didx[0]], add=True)
        wait_rows(rows1, sem1)
        wait_idx(2)
        wait_idx(3)
        plsc.subcore_barrier()
        for t in range(RPT // CH):
            pltpu.sync_copy(acc_sh.at[pl.ds(sid * RPT + t * CH, CH)], rows0)
            pltpu.sync_copy(rows0, out_hbm.at[cid, pl.ds(sid * RPT + t * CH, CH)])

    return k(hs, src_flat, dst_flat, zrow)


def _sc_predict(x2, ui_flat, ji_flat):
    @functools.partial(
        pl.kernel,
        mesh=_mesh(),
        out_type=(
            jax.ShapeDtypeStruct((B, D), jnp.float32),
            jax.ShapeDtypeStruct((B, D), jnp.float32),
        ),
        scratch_types=[
            pltpu.VMEM((PCH,), jnp.int32),
            pltpu.VMEM((PCH,), jnp.int32),
            pltpu.VMEM((PCH, D), jnp.float32),
            pltpu.VMEM((PCH, D), jnp.float32),
            pltpu.SemaphoreType.DMA,
            pltpu.SemaphoreType.DMA,
        ],
    )
    def k(x2_hbm, ui_hbm, ji_hbm, gu_hbm, gj_hbm, uix, jix, ub, jb, s0, s1):
        cid = lax.axis_index("c")
        sid = lax.axis_index("s")
        wid = cid * NS + sid

        def body(t, carry):
            pltpu.sync_copy(ui_hbm.at[pl.ds(wid * BPW + t * PCH, PCH)], uix)
            pltpu.sync_copy(ji_hbm.at[pl.ds(wid * BPW + t * PCH, PCH)], jix)
            cu = pltpu.async_copy(x2_hbm.at[uix], ub, s0)
            cj = pltpu.async_copy(x2_hbm.at[jix], jb, s1)
            cu.wait()
            cj.wait()
            pltpu.sync_copy(ub, gu_hbm.at[pl.ds(wid * BPW + t * PCH, PCH)])
            pltpu.sync_copy(jb, gj_hbm.at[pl.ds(wid * BPW + t * PCH, PCH)])
            return carry

        lax.fori_loop(0, NPC, body, 0)

    return k(x2, ui_flat, ji_flat)


def _tc_first(x, W1, degp):
    def body(x_ref, w_ref, degp_ref, hs_ref, dis_ref):
        deg = degp_ref[0, :N] + degp_ref[1, :N] + 1.0
        dis = 1.0 / jnp.sqrt(deg)
        h = jnp.dot(x_ref[...], w_ref[...], preferred_element_type=jnp.float32)
        hs_ref[...] = h * dis
        dis_ref[...] = dis

    return pl.pallas_call(
        body,
        out_shape=(
            jax.ShapeDtypeStruct((N, D), jnp.float32),
            jax.ShapeDtypeStruct((N, 1), jnp.float32),
        ),
    )(x, W1, degp)


def _tc_mid(aggp, hs1, dis, b1, W2):
    def body(aggp_ref, hs_ref, dis_ref, b_ref, w_ref, out_ref):
        agg = aggp_ref[0, :N] + aggp_ref[1, :N] + hs_ref[...]
        x1 = jnp.maximum(agg * dis_ref[...] + b_ref[...], 0.0)
        out_ref[...] = (
            jnp.dot(x1, w_ref[...], preferred_element_type=jnp.float32) * dis_ref[...]
        )

    return pl.pallas_call(
        body, out_shape=jax.ShapeDtypeStruct((N, D), jnp.float32)
    )(aggp, hs1, dis, b1, W2)


def _tc_final(aggp, hs2, dis, b2):
    def body(aggp_ref, hs_ref, dis_ref, b_ref, out_ref):
        out_ref[...] = (
            aggp_ref[0, :N] + aggp_ref[1, :N] + hs_ref[...]
        ) * dis_ref[...] + b_ref[...]

    return pl.pallas_call(
        body, out_shape=jax.ShapeDtypeStruct((N, D), jnp.float32)
    )(aggp, hs2, dis, b2)


def _tc_addpred(gu, gj, wpu, wpj):
    def body(gu_ref, gj_ref, wu_ref, wj_ref, out_ref):
        t = gu_ref[...] * wu_ref[...] + gj_ref[...] * wj_ref[...]
        out_ref[...] = jnp.sum(t, axis=1, keepdims=True)

    return pl.pallas_call(
        body, out_shape=jax.ShapeDtypeStruct((B, 1), jnp.float32)
    )(gu, gj, wpu, wpj)


def kernel(edge_index, user_indices, job_indices, user_emb, job_emb, W1, b1, W2, b2, Wp, bp):
    f32 = jnp.float32
    x = jnp.concatenate([user_emb, job_emb], axis=0)
    src_flat = edge_index[0]
    dst_flat = edge_index[1]
    zrow = jnp.zeros((CH, D), f32)
    ones_row = jnp.ones((CH, D), f32)

    degp = _sc_degree(dst_flat, ones_row, zrow)
    degp16 = degp[:, :N, :1]
    hs1, dis = _tc_first(x, W1, degp16)
    aggp1 = _sc_edge_agg(hs1, src_flat, dst_flat, zrow)
    hs2 = _tc_mid(aggp1, hs1, dis, b1.reshape(1, D), W2)
    aggp2 = _sc_edge_agg(hs2, src_flat, dst_flat, zrow)
    x2 = _tc_final(aggp2, hs2, dis, b2.reshape(1, D))
    gu, gj = _sc_predict(x2, user_indices, job_indices + NUM_USERS)
    wpu = Wp[:D, 0].reshape(1, D)
    wpj = Wp[D:, 0].reshape(1, D)
    pred = _tc_addpred(gu, gj, wpu, wpj)
    return pred.reshape(B) + bp[0]


# speedup vs baseline: 22.4883x; 1.0198x over previous
import functools

import jax
import jax.numpy as jnp
from jax import lax
from jax.experimental import pallas as pl
from jax.experimental.pallas import tpu as pltpu
from jax.experimental.pallas import tpu_sc as plsc

NUM_USERS = 5000
N = 10000
D = 128
E = 320000
B = 16384

NC = 2
NS = 16
NW = NC * NS
EPW = E // NW
CH = 80
NCH = EPW // CH
NPAD = 10240
RPT = NPAD // NS
BPW = B // NW
PCH = 128
NPC = BPW // PCH


def _mesh():
    return plsc.VectorSubcoreMesh(
        core_axis_name="c", subcore_axis_name="s", num_cores=NC, num_subcores=NS
    )


def _sc_degree(dst_flat, ones_row, zrow):
    @functools.partial(
        pl.kernel,
        mesh=_mesh(),
        out_type=jax.ShapeDtypeStruct((NC, NPAD, D), jnp.float32),
        scratch_types=[
            pltpu.VMEM((CH,), jnp.int32),
            pltpu.VMEM((CH,), jnp.int32),
            pltpu.VMEM((CH, D), jnp.float32),
            pltpu.VMEM_SHARED((NPAD, D), jnp.float32),
            pltpu.SemaphoreType.DMA,
        ],
    )
    def k(dst_hbm, ones_hbm, z_hbm, out_hbm, idx0, idx1, buf, acc_sh, sem):
        cid = lax.axis_index("c")
        sid = lax.axis_index("s")
        wid = cid * NS + sid
        base = wid * EPW
        pltpu.sync_copy(z_hbm, buf)
        for t in range(RPT // CH):
            pltpu.sync_copy(buf, acc_sh.at[pl.ds(sid * RPT + t * CH, CH)])
        pltpu.sync_copy(ones_hbm, buf)
        plsc.subcore_barrier()
        pltpu.sync_copy(dst_hbm.at[pl.ds(base, CH)], idx0)

        def body(i, carry):
            j1 = 2 * i + 1
            j2 = jnp.minimum(2 * i + 2, NCH - 1)
            pltpu.async_copy(dst_hbm.at[pl.ds(base + j1 * CH, CH)], idx1, sem)
            pltpu.sync_copy(buf, acc_sh.at[idx0], add=True)
            pltpu.make_async_copy(dst_hbm.at[pl.ds(base, CH)], idx1, sem).wait()
            pltpu.async_copy(dst_hbm.at[pl.ds(base + j2 * CH, CH)], idx0, sem)
            pltpu.sync_copy(buf, acc_sh.at[idx1], add=True)
            pltpu.make_async_copy(dst_hbm.at[pl.ds(base, CH)], idx0, sem).wait()
            return carry

        lax.fori_loop(0, (NCH - 1) // 2, body, 0)
        pltpu.sync_copy(buf, acc_sh.at[idx0], add=True)
        plsc.subcore_barrier()
        for t in range(RPT // CH):
            pltpu.sync_copy(acc_sh.at[pl.ds(sid * RPT + t * CH, CH)], buf)
            pltpu.sync_copy(buf, out_hbm.at[cid, pl.ds(sid * RPT + t * CH, CH)])

    return k(dst_flat, ones_row, zrow)


def _sc_edge_agg(hs, src_flat, dst_flat, zrow):
    @functools.partial(
        pl.kernel,
        mesh=_mesh(),
        out_type=jax.ShapeDtypeStruct((NC, NPAD, D), jnp.float32),
        scratch_types=[
            [pltpu.VMEM((CH,), jnp.int32)] * 4,
            [pltpu.VMEM((CH,), jnp.int32)] * 4,
            pltpu.VMEM((CH, D), jnp.float32),
            pltpu.VMEM((CH, D), jnp.float32),
            pltpu.VMEM_SHARED((NPAD, D), jnp.float32),
            pltpu.SemaphoreType.DMA,
            pltpu.SemaphoreType.DMA,
            [pltpu.SemaphoreType.DMA] * 4,
        ],
    )
    def k(hs_hbm, src_hbm, dst_hbm, z_hbm, out_hbm,
          sidx, didx, rows0, rows1, acc_sh, sem0, sem1, semi):
        cid = lax.axis_index("c")
        sid = lax.axis_index("s")
        wid = cid * NS + sid
        base = wid * EPW

        def load_idx(j, k, sem=None):
            if sem is None:
                pltpu.sync_copy(src_hbm.at[pl.ds(base + j * CH, CH)], sidx[k])
                pltpu.sync_copy(dst_hbm.at[pl.ds(base + j * CH, CH)], didx[k])
            else:
                pltpu.async_copy(src_hbm.at[pl.ds(base + j * CH, CH)], sidx[k], sem)
                pltpu.async_copy(dst_hbm.at[pl.ds(base + j * CH, CH)], didx[k], sem)

        def wait_idx(k):
            pltpu.make_async_copy(src_hbm.at[pl.ds(base, CH)], sidx[k], semi[k]).wait()
            pltpu.make_async_copy(dst_hbm.at[pl.ds(base, CH)], didx[k], semi[k]).wait()

        def wait_rows(buf, sem):
            pltpu.make_async_copy(hs_hbm.at[sidx[0]], buf, sem).wait()

        pltpu.sync_copy(z_hbm, rows0)
        for t in range(RPT // CH):
            pltpu.sync_copy(rows0, acc_sh.at[pl.ds(sid * RPT + t * CH, CH)])
        plsc.subcore_barrier()
        load_idx(0, 0)
        load_idx(1, 1)
        load_idx(2, 2, semi[2])
        load_idx(3, 3, semi[3])
        pltpu.async_copy(hs_hbm.at[sidx[0]], rows0, sem0)
        pltpu.async_copy(hs_hbm.at[sidx[1]], rows1, sem1)

        def body(i, carry):
            j4 = jnp.minimum(4 * i + 4, NCH - 1)
            j5 = jnp.minimum(4 * i + 5, NCH - 1)
            j6 = jnp.minimum(4 * i + 6, NCH - 1)
            j7 = jnp.minimum(4 * i + 7, NCH - 1)
            wait_rows(rows0, sem0)
            pltpu.sync_copy(rows0, acc_sh.at[didx[0]], add=True)
            load_idx(j4, 0, semi[0])
            wait_idx(2)
            pltpu.async_copy(hs_hbm.at[sidx[2]], rows0, sem0)
            wait_rows(rows1, sem1)
            pltpu.sync_copy(rows1, acc_sh.at[didx[1]], add=True)
            load_idx(j5, 1, semi[1])
            wait_idx(3)
            pltpu.async_copy(hs_hbm.at[sidx[3]], rows1, sem1)
            wait_rows(rows0, sem0)
            pltpu.sync_copy(rows0, acc_sh.at[didx[2]], add=True)
            load_idx(j6, 2, semi[2])
            wait_idx(0)
            pltpu.async_copy(hs_hbm.at[sidx[0]], rows0, sem0)
            wait_rows(rows1, sem1)
            pltpu.sync_copy(rows1, acc_sh.at[didx[3]], add=True)
            load_idx(j7, 3, semi[3])
            wait_idx(1)
            pltpu.async_copy(hs_hbm.at[sidx[1]], rows1, sem1)
            return carry

        lax.fori_loop(0, (NCH - 1) // 4, body, 0)
        wait_rows(rows0, sem0)
        pltpu.sync_copy(rows0, acc_sh.at[didx[0]], add=True)
        wait_rows(rows1, sem1)
        wait_idx(2)
        wait_idx(3)
        plsc.subcore_barrier()
        for t in range(RPT // CH):
            pltpu.sync_copy(acc_sh.at[pl.ds(sid * RPT + t * CH, CH)], rows0)
            pltpu.sync_copy(rows0, out_hbm.at[cid, pl.ds(sid * RPT + t * CH, CH)])

    return k(hs, src_flat, dst_flat, zrow)


def _sc_predict(x2, ui_flat, ji_flat):
    @functools.partial(
        pl.kernel,
        mesh=_mesh(),
        out_type=(
            jax.ShapeDtypeStruct((B, D), jnp.float32),
            jax.ShapeDtypeStruct((B, D), jnp.float32),
        ),
        scratch_types=[
            [pltpu.VMEM((PCH,), jnp.int32)] * NPC,
            [pltpu.VMEM((PCH,), jnp.int32)] * NPC,
            [pltpu.VMEM((PCH, D), jnp.float32)] * 2,
            [pltpu.VMEM((PCH, D), jnp.float32)] * 2,
            pltpu.SemaphoreType.DMA,
            pltpu.SemaphoreType.DMA,
        ],
    )
    def k(x2_hbm, ui_hbm, ji_hbm, gu_hbm, gj_hbm, uix, jix, ub, jb, s0, s1):
        cid = lax.axis_index("c")
        sid = lax.axis_index("s")
        wid = cid * NS + sid

        for t in range(NPC):
            pltpu.sync_copy(ui_hbm.at[pl.ds(wid * BPW + t * PCH, PCH)], uix[t])
            pltpu.sync_copy(ji_hbm.at[pl.ds(wid * BPW + t * PCH, PCH)], jix[t])
        cpu = [pltpu.async_copy(x2_hbm.at[uix[t]], ub[t], s0) for t in range(2)]
        cpj = [pltpu.async_copy(x2_hbm.at[jix[t]], jb[t], s1) for t in range(2)]
        for t in range(NPC):
            b = t % 2
            cpu[b].wait()
            pltpu.sync_copy(ub[b], gu_hbm.at[pl.ds(wid * BPW + t * PCH, PCH)])
            if t + 2 < NPC:
                cpu[b] = pltpu.async_copy(x2_hbm.at[uix[t + 2]], ub[b], s0)
            cpj[b].wait()
            pltpu.sync_copy(jb[b], gj_hbm.at[pl.ds(wid * BPW + t * PCH, PCH)])
            if t + 2 < NPC:
                cpj[b] = pltpu.async_copy(x2_hbm.at[jix[t + 2]], jb[b], s1)

    return k(x2, ui_flat, ji_flat)


def _tc_first(x, W1, degp):
    def body(x_ref, w_ref, degp_ref, hs_ref, dis_ref):
        deg = degp_ref[0, :N, :1] + degp_ref[1, :N, :1] + 1.0
        dis = 1.0 / jnp.sqrt(deg)
        h = jnp.dot(x_ref[...], w_ref[...], preferred_element_type=jnp.float32)
        hs_ref[...] = h * dis
        dis_ref[...] = dis

    return pl.pallas_call(
        body,
        out_shape=(
            jax.ShapeDtypeStruct((N, D), jnp.float32),
            jax.ShapeDtypeStruct((N, 1), jnp.float32),
        ),
    )(x, W1, degp)


def _tc_mid(aggp, hs1, dis, b1, W2):
    def body(aggp_ref, hs_ref, dis_ref, b_ref, w_ref, out_ref):
        agg = aggp_ref[0, :N] + aggp_ref[1, :N] + hs_ref[...]
        x1 = jnp.maximum(agg * dis_ref[...] + b_ref[...], 0.0)
        out_ref[...] = (
            jnp.dot(x1, w_ref[...], preferred_element_type=jnp.float32) * dis_ref[...]
        )

    return pl.pallas_call(
        body, out_shape=jax.ShapeDtypeStruct((N, D), jnp.float32)
    )(aggp, hs1, dis, b1, W2)


def _tc_final(aggp, hs2, dis, b2):
    def body(aggp_ref, hs_ref, dis_ref, b_ref, out_ref):
        out_ref[...] = (
            aggp_ref[0, :N] + aggp_ref[1, :N] + hs_ref[...]
        ) * dis_ref[...] + b_ref[...]

    return pl.pallas_call(
        body, out_shape=jax.ShapeDtypeStruct((N, D), jnp.float32)
    )(aggp, hs2, dis, b2)


def _tc_addpred(gu, gj, wpu, wpj):
    def body(gu_ref, gj_ref, wu_ref, wj_ref, out_ref):
        t = gu_ref[...] * wu_ref[...] + gj_ref[...] * wj_ref[...]
        out_ref[...] = jnp.sum(t, axis=1, keepdims=True)

    return pl.pallas_call(
        body, out_shape=jax.ShapeDtypeStruct((B, 1), jnp.float32)
    )(gu, gj, wpu, wpj)


def kernel(edge_index, user_indices, job_indices, user_emb, job_emb, W1, b1, W2, b2, Wp, bp):
    f32 = jnp.float32
    x = jnp.concatenate([user_emb, job_emb], axis=0)
    src_flat = edge_index[0]
    dst_flat = edge_index[1]
    zrow = jnp.zeros((CH, D), f32)
    ones_row = jnp.ones((CH, D), f32)

    degp = _sc_degree(dst_flat, ones_row, zrow)
    hs1, dis = _tc_first(x, W1, degp)
    aggp1 = _sc_edge_agg(hs1, src_flat, dst_flat, zrow)
    hs2 = _tc_mid(aggp1, hs1, dis, b1.reshape(1, D), W2)
    aggp2 = _sc_edge_agg(hs2, src_flat, dst_flat, zrow)
    x2 = _tc_final(aggp2, hs2, dis, b2.reshape(1, D))
    gu, gj = _sc_predict(x2, user_indices, job_indices + NUM_USERS)
    wpu = Wp[:D, 0].reshape(1, D)
    wpj = Wp[D:, 0].reshape(1, D)
    pred = _tc_addpred(gu, gj, wpu, wpj)
    return pred.reshape(B) + bp[0]
